# baseline (speedup 1.0000x reference)
import functools

import jax
import jax.numpy as jnp
from jax import lax
from jax.experimental import pallas as pl
from jax.experimental.pallas import tpu as pltpu
from jax.experimental.pallas import tpu_sc as plsc

VOCAB = 100000
D = 128
B = 16384

NC = 2
NS = 16
NW = NC * NS
BPW = B // NW
CHUNK = 128
NCH = BPW // CHUNK

_mesh = plsc.VectorSubcoreMesh(core_axis_name="c", subcore_axis_name="s")


@functools.partial(
    pl.kernel,
    out_type=jax.ShapeDtypeStruct((B, D), jnp.float32),
    mesh=_mesh,
    scratch_types=[
        pltpu.VMEM((NCH, CHUNK), jnp.int32),
        pltpu.VMEM((BPW, D), jnp.float32),
        pltpu.VMEM_SHARED((NS * 2 * CHUNK, D), jnp.float32),
        pltpu.SemaphoreType.DMA((NCH,)),
        pltpu.SemaphoreType.DMA((NCH,)),
        pltpu.SemaphoreType.DMA((NCH,)),
        pltpu.SemaphoreType.DMA((NCH,)),
    ],
)
def _embed_sc(idx_hbm, table_hbm, out_hbm, idx_v, rows_v, rows_sh,
              isem, gsem, xsem, osem):
    wid = lax.axis_index("s") * NC + lax.axis_index("c")
    sid = lax.axis_index("s")
    base = wid * BPW
    sbase = sid * 2 * CHUNK
    idx_copies = [
        pltpu.async_copy(idx_hbm.at[wid, j], idx_v.at[j], isem.at[j])
        for j in range(NCH)
    ]
    gathers = []
    for j in range(NCH):
        idx_copies[j].wait()
        gathers.append(
            pltpu.async_copy(
                table_hbm.at[idx_v.at[j]],
                rows_v.at[pl.ds(j * CHUNK, CHUNK)],
                gsem.at[j],
            )
        )
    outs = []
    for j in range(NCH):
        gathers[j].wait()
        if j >= 2:
            outs[j - 2].wait()
        slot = sbase + (j % 2) * CHUNK
        pltpu.async_copy(
            rows_v.at[pl.ds(j * CHUNK, CHUNK)],
            rows_sh.at[pl.ds(slot, CHUNK)],
            xsem.at[j],
        ).wait()
        outs.append(
            pltpu.async_copy(
                rows_sh.at[pl.ds(slot, CHUNK)],
                out_hbm.at[pl.ds(base + j * CHUNK, CHUNK)],
                osem.at[j],
            )
        )
    outs[NCH - 2].wait()
    outs[NCH - 1].wait()


def kernel(x, weight):
    idx = x.astype(jnp.int32).reshape(NW, NCH, CHUNK)
    return _embed_sc(idx, weight)


# speedup vs baseline: 1.0954x; 1.0954x over previous
import functools

import jax
import jax.numpy as jnp
from jax import lax
from jax.experimental import pallas as pl
from jax.experimental.pallas import tpu as pltpu
from jax.experimental.pallas import tpu_sc as plsc

VOCAB = 100000
D = 128
B = 16384

NC = 2
NS = 16
NW = NC * NS
BPW = B // NW
CHUNK = 128
NCH = BPW // CHUNK

_mesh = plsc.VectorSubcoreMesh(core_axis_name="c", subcore_axis_name="s")


@functools.partial(
    pl.kernel,
    out_type=jax.ShapeDtypeStruct((B, D), jnp.float32),
    mesh=_mesh,
    scratch_types=[
        pltpu.VMEM((NCH, CHUNK), jnp.int32),
        pltpu.VMEM((BPW, D), jnp.float32),
        pltpu.SemaphoreType.DMA((NCH,)),
        pltpu.SemaphoreType.DMA,
    ],
)
def _embed_sc(idx_hbm, table_hbm, out_hbm, idx_v, rows_v, isem, gsem):
    wid = lax.axis_index("s") * NC + lax.axis_index("c")
    base = wid * BPW
    idx_copies = [
        pltpu.async_copy(idx_hbm.at[wid, j], idx_v.at[j], isem.at[j])
        for j in range(NCH)
    ]
    gathers = []
    for j in range(NCH):
        idx_copies[j].wait()
        gathers.append(
            pltpu.async_copy(
                table_hbm.at[idx_v.at[j]],
                rows_v.at[pl.ds(j * CHUNK, CHUNK)],
                gsem,
            )
        )
    for c in gathers:
        c.wait()
    pltpu.sync_copy(rows_v, out_hbm.at[pl.ds(base, BPW)])


def kernel(x, weight):
    idx = x.astype(jnp.int32).reshape(NW, NCH, CHUNK)
    return _embed_sc(idx, weight)
